# baseline (speedup 1.0000x reference)
import functools

import jax
import jax.numpy as jnp
from jax import lax
from jax.experimental import pallas as pl
from jax.experimental.pallas import tpu as pltpu
from jax.experimental.pallas import tpu_sc as plsc

_NSEL = 64
_STRIDE = 1000
_COLS = 512
_L = 16
_NC = 2
_NS = 16


def _sc_fix_body(xref, buf):
    c = lax.axis_index("c")
    s = lax.axis_index("s")
    w = s * _NC + c
    for j in range(2):
        row = (2 * w + j) * _STRIDE
        pltpu.sync_copy(xref.at[row], buf.at[j])
        for t in range(_COLS // _L):
            v = buf[j, pl.ds(t * _L, _L)]
            buf[j, pl.ds(t * _L, _L)] = jnp.maximum(v, 0.0)
        pltpu.sync_copy(buf.at[j], xref.at[row])


@functools.cache
def _sc_fix():
    mesh = plsc.VectorSubcoreMesh(
        core_axis_name="c", subcore_axis_name="s", num_cores=_NC, num_subcores=_NS
    )
    return pl.kernel(
        _sc_fix_body,
        out_type=(),
        mesh=mesh,
        scratch_types=[pltpu.VMEM((2, _COLS), jnp.float32)],
    )


def kernel(x):
    ref = jax.new_ref(x)
    _sc_fix()(ref)
    return jax.freeze(ref)
